# initial kernel scaffold (unmeasured)
import jax
import jax.numpy as jnp
from jax import lax
from jax.experimental import pallas as pl
from jax.experimental.pallas import tpu as pltpu

TILE = 512


def kernel(x, W):
    T, D = x.shape
    _, V_loc = W.shape
    V = 2 * V_loc
    n_loc = V_loc // TILE
    n_all = V // TILE

    def body(x_ref, w_ref, out_ref, wtile, ltile, ms_send, ms_recv,
             wsem, osem, send_sems, recv_sems, ms_sems):
        my_x = lax.axis_index("x")
        my_y = lax.axis_index("y")
        nbr = (1 - my_x, my_y)
        col0 = my_x * V_loc

        barrier_sem = pltpu.get_barrier_semaphore()
        pl.semaphore_signal(barrier_sem, inc=1, device_id=nbr,
                            device_id_type=pl.DeviceIdType.MESH)
        pl.semaphore_wait(barrier_sem, 1)

        def tile_step(t, slot, m, s):
            cpw = pltpu.make_async_copy(
                w_ref.at[:, pl.ds(t * TILE, TILE)], wtile.at[slot],
                wsem.at[slot])
            cpw.start()
            cpw.wait()
            logits = jnp.dot(x_ref[:, :], wtile[slot],
                             preferred_element_type=jnp.float32)
            ltile[slot] = logits
            tm = jnp.max(logits, axis=1, keepdims=True)
            m_new = jnp.maximum(m, tm)
            s_new = s * jnp.exp(m - m_new) + jnp.sum(
                jnp.exp(logits - m_new), axis=1, keepdims=True)
            cpo = pltpu.make_async_copy(
                ltile.at[slot], out_ref.at[:, pl.ds(col0 + t * TILE, TILE)],
                osem.at[slot])
            cpo.start()
            rdma = pltpu.make_async_remote_copy(
                src_ref=ltile.at[slot],
                dst_ref=out_ref.at[:, pl.ds(col0 + t * TILE, TILE)],
                send_sem=send_sems.at[slot],
                recv_sem=recv_sems.at[slot],
                device_id=nbr,
                device_id_type=pl.DeviceIdType.MESH,
            )
            rdma.start()
            cpo.wait()
            rdma.wait()
            return m_new, s_new

        def pass1_body(i, carry):
            m, s = carry
            m, s = tile_step(2 * i, 0, m, s)
            m, s = tile_step(2 * i + 1, 1, m, s)
            return m, s

        m0 = jnp.full((T, 1), -1e30, jnp.float32)
        s0 = jnp.zeros((T, 1), jnp.float32)
        m, s = lax.fori_loop(0, n_loc // 2, pass1_body, (m0, s0))

        ms_send[:, 0:128] = jnp.broadcast_to(m, (T, 128))
        ms_send[:, 128:256] = jnp.broadcast_to(s, (T, 128))
        ms_rdma = pltpu.make_async_remote_copy(
            src_ref=ms_send, dst_ref=ms_recv,
            send_sem=ms_sems.at[0], recv_sem=ms_sems.at[1],
            device_id=nbr, device_id_type=pl.DeviceIdType.MESH)
        ms_rdma.start()
        ms_rdma.wait()
        m_n = ms_recv[:, 0:1]
        s_n = ms_recv[:, 128:129]
        m_g = jnp.maximum(m, m_n)
        s_g = s * jnp.exp(m - m_g) + s_n * jnp.exp(m_n - m_g)
        inv_s = 1.0 / s_g

        def pass2_body(t, carry):
            cin = pltpu.make_async_copy(
                out_ref.at[:, pl.ds(t * TILE, TILE)], ltile.at[0],
                wsem.at[0])
            cin.start()
            cin.wait()
            ltile[1] = jnp.exp(ltile[0] - m_g) * inv_s
            cout = pltpu.make_async_copy(
                ltile.at[1], out_ref.at[:, pl.ds(t * TILE, TILE)],
                wsem.at[1])
            cout.start()
            cout.wait()
            return carry

        lax.fori_loop(0, n_all, pass2_body, 0)

    return pl.pallas_call(
        body,
        out_shape=jax.ShapeDtypeStruct((T, V), jnp.float32),
        in_specs=[
            pl.BlockSpec(memory_space=pltpu.VMEM),
            pl.BlockSpec(memory_space=pltpu.ANY),
        ],
        out_specs=pl.BlockSpec(memory_space=pltpu.ANY),
        scratch_shapes=[
            pltpu.VMEM((2, D, TILE), jnp.float32),
            pltpu.VMEM((2, T, TILE), jnp.float32),
            pltpu.VMEM((T, 256), jnp.float32),
            pltpu.VMEM((T, 256), jnp.float32),
            pltpu.SemaphoreType.DMA((2,)),
            pltpu.SemaphoreType.DMA((2,)),
            pltpu.SemaphoreType.DMA((2,)),
            pltpu.SemaphoreType.DMA((2,)),
            pltpu.SemaphoreType.DMA((2,)),
        ],
        compiler_params=pltpu.CompilerParams(collective_id=0),
    )(x, W)


# baseline (device time: 1256465 ns/iter reference)
import jax
import jax.numpy as jnp
from jax import lax
from jax.experimental import pallas as pl
from jax.experimental.pallas import tpu as pltpu

TILE = 512


def kernel(x, W):
    T, D = x.shape
    _, V_loc = W.shape
    V = 2 * V_loc
    n_loc = V_loc // TILE
    n_all = V // TILE

    def body(x_ref, w_ref, out_ref, wtile, ltile, ms_send, ms_recv,
             wsem, osem, send_sems, recv_sems, ms_sems):
        my_x = lax.axis_index("x")
        my_y = lax.axis_index("y")
        nbr = (1 - my_x, my_y)
        col0 = my_x * V_loc

        barrier_sem = pltpu.get_barrier_semaphore()
        pl.semaphore_signal(barrier_sem, inc=1, device_id=nbr,
                            device_id_type=pl.DeviceIdType.MESH)
        pl.semaphore_wait(barrier_sem, 1)

        def tile_step(t, slot, m, s):
            cpw = pltpu.make_async_copy(
                w_ref.at[:, pl.ds(t * TILE, TILE)], wtile.at[slot],
                wsem.at[slot])
            cpw.start()
            cpw.wait()
            logits = jnp.dot(x_ref[:, :], wtile[slot],
                             preferred_element_type=jnp.float32)
            ltile[slot] = logits
            tm = jnp.max(logits, axis=1, keepdims=True)
            m_new = jnp.maximum(m, tm)
            s_new = s * jnp.exp(m - m_new) + jnp.sum(
                jnp.exp(logits - m_new), axis=1, keepdims=True)
            cpo = pltpu.make_async_copy(
                ltile.at[slot], out_ref.at[:, pl.ds(col0 + t * TILE, TILE)],
                osem.at[slot])
            cpo.start()
            rdma = pltpu.make_async_remote_copy(
                src_ref=ltile.at[slot],
                dst_ref=out_ref.at[:, pl.ds(col0 + t * TILE, TILE)],
                send_sem=send_sems.at[slot],
                recv_sem=recv_sems.at[slot],
                device_id=nbr,
                device_id_type=pl.DeviceIdType.MESH,
            )
            rdma.start()
            cpo.wait()
            rdma.wait()
            return m_new, s_new

        def pass1_body(i, carry):
            m, s = carry
            m, s = tile_step(2 * i, 0, m, s)
            m, s = tile_step(2 * i + 1, 1, m, s)
            return m, s

        m0 = jnp.full((T, 1), -1e30, jnp.float32)
        s0 = jnp.zeros((T, 1), jnp.float32)
        m, s = lax.fori_loop(0, n_loc // 2, pass1_body, (m0, s0))

        ms_send[:, 0:128] = jnp.broadcast_to(m, (T, 128))
        ms_send[:, 128:256] = jnp.broadcast_to(s, (T, 128))
        ms_rdma = pltpu.make_async_remote_copy(
            src_ref=ms_send, dst_ref=ms_recv,
            send_sem=ms_sems.at[0], recv_sem=ms_sems.at[1],
            device_id=nbr, device_id_type=pl.DeviceIdType.MESH)
        ms_rdma.start()
        ms_rdma.wait()
        m_n = ms_recv[:, 0:1]
        s_n = ms_recv[:, 128:129]
        m_g = jnp.maximum(m, m_n)
        s_g = s * jnp.exp(m - m_g) + s_n * jnp.exp(m_n - m_g)
        inv_s = 1.0 / s_g

        def pass2_body(t, carry):
            cin = pltpu.make_async_copy(
                out_ref.at[:, pl.ds(t * TILE, TILE)], ltile.at[0],
                wsem.at[0])
            cin.start()
            cin.wait()
            ltile[1] = jnp.exp(ltile[0] - m_g) * inv_s
            cout = pltpu.make_async_copy(
                ltile.at[1], out_ref.at[:, pl.ds(t * TILE, TILE)],
                wsem.at[1])
            cout.start()
            cout.wait()
            return carry

        lax.fori_loop(0, n_all, pass2_body, 0)

    return pl.pallas_call(
        body,
        out_shape=jax.ShapeDtypeStruct((T, V), jnp.float32),
        in_specs=[
            pl.BlockSpec(memory_space=pltpu.VMEM),
            pl.BlockSpec(memory_space=pl.ANY),
        ],
        out_specs=pl.BlockSpec(memory_space=pl.ANY),
        scratch_shapes=[
            pltpu.VMEM((2, D, TILE), jnp.float32),
            pltpu.VMEM((2, T, TILE), jnp.float32),
            pltpu.VMEM((T, 256), jnp.float32),
            pltpu.VMEM((T, 256), jnp.float32),
            pltpu.SemaphoreType.DMA((2,)),
            pltpu.SemaphoreType.DMA((2,)),
            pltpu.SemaphoreType.DMA((2,)),
            pltpu.SemaphoreType.DMA((2,)),
            pltpu.SemaphoreType.DMA((2,)),
        ],
        compiler_params=pltpu.CompilerParams(collective_id=0),
    )(x, W)


# device time: 991197 ns/iter; 1.2676x vs baseline; 1.2676x over previous
import jax
import jax.numpy as jnp
from jax import lax
from jax.experimental import pallas as pl
from jax.experimental.pallas import tpu as pltpu

TILE = 512


def kernel(x, W):
    T, D = x.shape
    _, V_loc = W.shape
    V = 2 * V_loc
    n_loc = V_loc // TILE
    n_all = V // TILE

    def body(x_ref, w_ref, out_ref, wtile, ltile, otile, ms_send, ms_recv,
             lsem, osem, send_sems, recv_sems, ms_sems):
        my_x = lax.axis_index("x")
        my_y = lax.axis_index("y")
        nbr = (1 - my_x, my_y)
        col0 = my_x * V_loc

        def w_load(t, slot):
            return pltpu.make_async_copy(
                w_ref.at[:, pl.ds(t * TILE, TILE)], wtile.at[slot],
                lsem.at[slot])

        def out_store(t, slot, src):
            return pltpu.make_async_copy(
                src.at[slot], out_ref.at[:, pl.ds(col0 + t * TILE, TILE)],
                osem.at[slot])

        def rdma(t, slot):
            return pltpu.make_async_remote_copy(
                src_ref=ltile.at[slot],
                dst_ref=out_ref.at[:, pl.ds(col0 + t * TILE, TILE)],
                send_sem=send_sems.at[slot],
                recv_sem=recv_sems.at[slot],
                device_id=nbr,
                device_id_type=pl.DeviceIdType.MESH,
            )

        barrier_sem = pltpu.get_barrier_semaphore()
        pl.semaphore_signal(barrier_sem, inc=1, device_id=nbr,
                            device_id_type=pl.DeviceIdType.MESH)
        pl.semaphore_wait(barrier_sem, 1)

        w_load(0, 0).start()

        def p1_body(i, carry):
            m, s = carry
            t0 = 2 * i
            t1 = 2 * i + 1

            w_load(t0, 0).wait()
            w_load(t1, 1).start()
            logits = jnp.dot(x_ref[:, :], wtile[0],
                             preferred_element_type=jnp.float32)
            tm = jnp.max(logits, axis=1, keepdims=True)
            m1 = jnp.maximum(m, tm)
            s1 = s * jnp.exp(m - m1) + jnp.sum(
                jnp.exp(logits - m1), axis=1, keepdims=True)

            @pl.when(i >= 1)
            def _():
                rdma(t0 - 1, 1).wait()
                out_store(t0 - 1, 1, ltile).wait()

            ltile[0] = logits
            out_store(t0, 0, ltile).start()
            rdma(t0, 0).start()

            w_load(t1, 1).wait()

            @pl.when(i < n_loc // 2 - 1)
            def _():
                w_load(t1 + 1, 0).start()

            logits = jnp.dot(x_ref[:, :], wtile[1],
                             preferred_element_type=jnp.float32)
            tm = jnp.max(logits, axis=1, keepdims=True)
            m2 = jnp.maximum(m1, tm)
            s2 = s1 * jnp.exp(m1 - m2) + jnp.sum(
                jnp.exp(logits - m2), axis=1, keepdims=True)

            rdma(t0, 0).wait()
            out_store(t0, 0, ltile).wait()

            ltile[1] = logits
            out_store(t1, 1, ltile).start()
            rdma(t1, 1).start()
            return m2, s2

        m0 = jnp.full((T, 1), -1e30, jnp.float32)
        s0 = jnp.zeros((T, 1), jnp.float32)
        m, s = lax.fori_loop(0, n_loc // 2, p1_body, (m0, s0))
        rdma(n_loc - 1, 1).wait()
        out_store(n_loc - 1, 1, ltile).wait()

        ms_send[:, 0:128] = jnp.broadcast_to(m, (T, 128))
        ms_send[:, 128:256] = jnp.broadcast_to(s, (T, 128))
        ms_rdma = pltpu.make_async_remote_copy(
            src_ref=ms_send, dst_ref=ms_recv,
            send_sem=ms_sems.at[0], recv_sem=ms_sems.at[1],
            device_id=nbr, device_id_type=pl.DeviceIdType.MESH)
        ms_rdma.start()
        ms_rdma.wait()
        m_n = ms_recv[:, 0:1]
        s_n = ms_recv[:, 128:129]
        m_g = jnp.maximum(m, m_n)
        s_g = s * jnp.exp(m - m_g) + s_n * jnp.exp(m_n - m_g)
        inv_s = 1.0 / s_g

        def p2_in(t, slot):
            return pltpu.make_async_copy(
                out_ref.at[:, pl.ds(t * TILE, TILE)], ltile.at[slot],
                lsem.at[slot])

        def p2_out(t, slot):
            return pltpu.make_async_copy(
                otile.at[slot], out_ref.at[:, pl.ds(t * TILE, TILE)],
                osem.at[slot])

        p2_in(0, 0).start()
        p2_in(1, 1).start()

        def p2_sub(i, t, slot):
            p2_in(t, slot).wait()
            val = jnp.exp(ltile[slot] - m_g) * inv_s

            @pl.when(i >= 1)
            def _():
                p2_out(t - 2, slot).wait()

            otile[slot] = val
            p2_out(t, slot).start()

            @pl.when(t + 2 < n_all)
            def _():
                p2_in(t + 2, slot).start()

        def p2_body(i, carry):
            p2_sub(i, 2 * i, 0)
            p2_sub(i, 2 * i + 1, 1)
            return carry

        lax.fori_loop(0, n_all // 2, p2_body, 0)
        p2_out(n_all - 2, 0).wait()
        p2_out(n_all - 1, 1).wait()

    return pl.pallas_call(
        body,
        out_shape=jax.ShapeDtypeStruct((T, V), jnp.float32),
        in_specs=[
            pl.BlockSpec(memory_space=pltpu.VMEM),
            pl.BlockSpec(memory_space=pl.ANY),
        ],
        out_specs=pl.BlockSpec(memory_space=pl.ANY),
        scratch_shapes=[
            pltpu.VMEM((2, D, TILE), jnp.float32),
            pltpu.VMEM((2, T, TILE), jnp.float32),
            pltpu.VMEM((2, T, TILE), jnp.float32),
            pltpu.VMEM((T, 256), jnp.float32),
            pltpu.VMEM((T, 256), jnp.float32),
            pltpu.SemaphoreType.DMA((2,)),
            pltpu.SemaphoreType.DMA((2,)),
            pltpu.SemaphoreType.DMA((2,)),
            pltpu.SemaphoreType.DMA((2,)),
            pltpu.SemaphoreType.DMA((2,)),
        ],
        compiler_params=pltpu.CompilerParams(collective_id=0),
    )(x, W)


# device time: 658621 ns/iter; 1.9077x vs baseline; 1.5050x over previous
import jax
import jax.numpy as jnp
from jax import lax
from jax.experimental import pallas as pl
from jax.experimental.pallas import tpu as pltpu

TILE = 512
NSEND = 16
NSLOT = 12


def kernel(x, W):
    T, D = x.shape
    _, V_loc = W.shape
    V = 2 * V_loc
    n_loc = V_loc // TILE
    n_all = V // TILE

    def body(x_ref, w_ref, out_ref, wtile, send_ring, fast_ring, otile,
             mstat, sstat, ms_send, ms_recv, scale_ref,
             lsem, osem, cposem, xs_sems, xr_sems, fs_sems, fr_sems,
             ms_sems):
        my_x = lax.axis_index("x")
        my_y = lax.axis_index("y")
        nbr_x = (1 - my_x, my_y)
        nbr_y = (my_x, 1 - my_y)
        col0 = my_x * V_loc
        nbr_col0 = (1 - my_x) * V_loc
        q0 = 16 * my_y

        def tile(j):
            return q0 + j if j < NSEND else j - q0

        def w_load(j, slot):
            return pltpu.make_async_copy(
                w_ref.at[:, pl.ds(tile(j) * TILE, TILE)], wtile.at[slot],
                lsem.at[slot])

        def cpo(j, ring, slot):
            return pltpu.make_async_copy(
                ring.at[slot],
                out_ref.at[:, pl.ds(col0 + tile(j) * TILE, TILE)],
                cposem.at[j])

        def xsend(j):
            return pltpu.make_async_remote_copy(
                src_ref=send_ring.at[j % NSLOT],
                dst_ref=out_ref.at[:, pl.ds(col0 + tile(j) * TILE, TILE)],
                send_sem=xs_sems.at[j],
                recv_sem=xr_sems.at[j],
                device_id=nbr_x,
                device_id_type=pl.DeviceIdType.MESH,
            )

        def xrecv(f):
            sl = out_ref.at[:, pl.ds(nbr_col0 + (q0 + f) * TILE, TILE)]
            return pltpu.make_async_remote_copy(
                src_ref=sl, dst_ref=sl,
                send_sem=xs_sems.at[f], recv_sem=xr_sems.at[f],
                device_id=nbr_x, device_id_type=pl.DeviceIdType.MESH,
            )

        def fwd(f):
            sl = out_ref.at[:, pl.ds(nbr_col0 + (q0 + f) * TILE, TILE)]
            return pltpu.make_async_remote_copy(
                src_ref=sl, dst_ref=sl,
                send_sem=fs_sems.at[f], recv_sem=fr_sems.at[f],
                device_id=nbr_y, device_id_type=pl.DeviceIdType.MESH,
            )

        def frecv(f):
            sl = out_ref.at[:, pl.ds(nbr_col0 + (16 - q0 + f) * TILE, TILE)]
            return pltpu.make_async_remote_copy(
                src_ref=sl, dst_ref=sl,
                send_sem=fs_sems.at[f], recv_sem=fr_sems.at[f],
                device_id=nbr_y, device_id_type=pl.DeviceIdType.MESH,
            )

        barrier_sem = pltpu.get_barrier_semaphore()
        for nb in (nbr_x, nbr_y):
            pl.semaphore_signal(barrier_sem, inc=1, device_id=nb,
                                device_id_type=pl.DeviceIdType.MESH)
        pl.semaphore_wait(barrier_sem, 2)

        w_load(0, 0).start()
        for j in range(n_loc):
            if j >= NSEND:
                f = j - NSEND
                xrecv(f).wait_recv()
                fwd(f).start()

            w_load(j, j % 2).wait()
            if j + 1 < n_loc:
                w_load(j + 1, (j + 1) % 2).start()
            logits = jnp.dot(x_ref[:, :], wtile[j % 2],
                             preferred_element_type=jnp.float32)
            mt = jnp.max(logits, axis=1, keepdims=True)
            e = jnp.exp(logits - mt)
            st = jnp.sum(e, axis=1, keepdims=True)
            mstat[:, j:j + 1] = mt
            sstat[:, j:j + 1] = st

            if j < NSEND:
                slot = j % NSLOT
                if j >= NSLOT:
                    xsend(j - NSLOT).wait_send()
                    cpo(j - NSLOT, send_ring, slot).wait()
                send_ring[slot] = e
                cpo(j, send_ring, slot).start()
                xsend(j).start()
            else:
                slot = j % 2
                if j >= NSEND + 2:
                    cpo(j - 2, fast_ring, slot).wait()
                fast_ring[slot] = e
                cpo(j, fast_ring, slot).start()

        for j in range(NSEND - NSLOT, NSEND):
            xsend(j).wait_send()
        for j in range(NSEND - NSLOT, NSEND):
            cpo(j, send_ring, j % NSLOT).wait()
        for j in range(n_loc - 2, n_loc):
            cpo(j, fast_ring, j % 2).wait()
        for f in range(NSEND):
            fwd(f).wait_send()
        for f in range(NSEND):
            frecv(f).wait_recv()

        ms_send[:, 0:128] = mstat[:, :]
        ms_send[:, 128:256] = sstat[:, :]
        ms_rdma = pltpu.make_async_remote_copy(
            src_ref=ms_send, dst_ref=ms_recv,
            send_sem=ms_sems.at[0], recv_sem=ms_sems.at[1],
            device_id=nbr_x, device_id_type=pl.DeviceIdType.MESH)
        ms_rdma.start()
        ms_rdma.wait()

        m_mine_j = ms_send[:, 0:32]
        s_mine_j = ms_send[:, 128:160]
        m_nbr_j = ms_recv[:, 0:32]
        s_nbr_j = ms_recv[:, 128:160]
        m_all = jnp.maximum(jnp.max(m_mine_j, axis=1, keepdims=True),
                            jnp.max(m_nbr_j, axis=1, keepdims=True))
        s_all = (jnp.sum(jnp.exp(m_mine_j - m_all) * s_mine_j,
                         axis=1, keepdims=True)
                 + jnp.sum(jnp.exp(m_nbr_j - m_all) * s_nbr_j,
                           axis=1, keepdims=True))
        inv_s = 1.0 / s_all

        def by_tile(a):
            swapped = jnp.concatenate([a[:, 16:32], a[:, 0:16]], axis=1)
            return jnp.where(my_y == 0, a, swapped)

        scale_mine = jnp.exp(by_tile(m_mine_j) - m_all) * inv_s
        scale_nbr = jnp.exp(by_tile(m_nbr_j) - m_all) * inv_s
        is_x0 = (my_x == 0)
        scale_ref[:, 0:32] = jnp.where(is_x0, scale_mine, scale_nbr)
        scale_ref[:, 32:64] = jnp.where(is_x0, scale_nbr, scale_mine)

        def p2_in(g, slot):
            return pltpu.make_async_copy(
                out_ref.at[:, pl.ds(g * TILE, TILE)], fast_ring.at[slot],
                lsem.at[slot])

        def p2_out(g, slot):
            return pltpu.make_async_copy(
                otile.at[slot], out_ref.at[:, pl.ds(g * TILE, TILE)],
                osem.at[slot])

        p2_in(0, 0).start()
        p2_in(1, 1).start()
        for g in range(n_all):
            slot = g % 2
            p2_in(g, slot).wait()
            val = fast_ring[slot] * scale_ref[:, g:g + 1]
            if g >= 2:
                p2_out(g - 2, slot).wait()
            otile[slot] = val
            p2_out(g, slot).start()
            if g + 2 < n_all:
                p2_in(g + 2, slot).start()
        p2_out(n_all - 2, 0).wait()
        p2_out(n_all - 1, 1).wait()

    return pl.pallas_call(
        body,
        out_shape=jax.ShapeDtypeStruct((T, V), jnp.float32),
        in_specs=[
            pl.BlockSpec(memory_space=pltpu.VMEM),
            pl.BlockSpec(memory_space=pl.ANY),
        ],
        out_specs=pl.BlockSpec(memory_space=pl.ANY),
        scratch_shapes=[
            pltpu.VMEM((2, D, TILE), jnp.float32),
            pltpu.VMEM((NSLOT, T, TILE), jnp.float32),
            pltpu.VMEM((2, T, TILE), jnp.float32),
            pltpu.VMEM((2, T, TILE), jnp.float32),
            pltpu.VMEM((T, 128), jnp.float32),
            pltpu.VMEM((T, 128), jnp.float32),
            pltpu.VMEM((T, 256), jnp.float32),
            pltpu.VMEM((T, 256), jnp.float32),
            pltpu.VMEM((T, 128), jnp.float32),
            pltpu.SemaphoreType.DMA((2,)),
            pltpu.SemaphoreType.DMA((2,)),
            pltpu.SemaphoreType.DMA((n_loc,)),
            pltpu.SemaphoreType.DMA((NSEND,)),
            pltpu.SemaphoreType.DMA((NSEND,)),
            pltpu.SemaphoreType.DMA((NSEND,)),
            pltpu.SemaphoreType.DMA((NSEND,)),
            pltpu.SemaphoreType.DMA((2,)),
        ],
        compiler_params=pltpu.CompilerParams(
            collective_id=0, vmem_limit_bytes=96 * 1024 * 1024),
    )(x, W)


# device time: 599093 ns/iter; 2.0973x vs baseline; 1.0994x over previous
import jax
import jax.numpy as jnp
from jax import lax
from jax.experimental import pallas as pl
from jax.experimental.pallas import tpu as pltpu

TILE = 512
NSEND = 16
NSLOT = 12


def kernel(x, W):
    T, D = x.shape
    _, V_loc = W.shape
    V = 2 * V_loc
    n_loc = V_loc // TILE
    n_all = V // TILE

    def body(x_ref, w_ref, out_ref, wtile, send_ring, fast_ring, otile,
             mstat, sstat, ms_send, ms_recv, scale_ref,
             lsem, osem, cposem, xs_sems, xr_sems, fs_sems, fr_sems,
             ms_sems):
        my_x = lax.axis_index("x")
        my_y = lax.axis_index("y")
        nbr_x = (1 - my_x, my_y)
        nbr_y = (my_x, 1 - my_y)
        col0 = my_x * V_loc
        nbr_col0 = (1 - my_x) * V_loc
        q0 = 16 * my_y

        def tile(j):
            return q0 + j if j < NSEND else j - q0

        def w_load(j, slot):
            return pltpu.make_async_copy(
                w_ref.at[:, pl.ds(tile(j) * TILE, TILE)], wtile.at[slot],
                lsem.at[slot])

        def cpo(j, ring, slot):
            return pltpu.make_async_copy(
                ring.at[slot],
                out_ref.at[:, pl.ds(col0 + tile(j) * TILE, TILE)],
                cposem.at[j])

        def xsend(j):
            return pltpu.make_async_remote_copy(
                src_ref=send_ring.at[j % NSLOT],
                dst_ref=out_ref.at[:, pl.ds(col0 + tile(j) * TILE, TILE)],
                send_sem=xs_sems.at[j],
                recv_sem=xr_sems.at[j],
                device_id=nbr_x,
                device_id_type=pl.DeviceIdType.MESH,
            )

        def xrecv(f):
            sl = out_ref.at[:, pl.ds(nbr_col0 + (q0 + f) * TILE, TILE)]
            return pltpu.make_async_remote_copy(
                src_ref=sl, dst_ref=sl,
                send_sem=xs_sems.at[f], recv_sem=xr_sems.at[f],
                device_id=nbr_x, device_id_type=pl.DeviceIdType.MESH,
            )

        def fwd(f):
            sl = out_ref.at[:, pl.ds(nbr_col0 + (q0 + f) * TILE, TILE)]
            return pltpu.make_async_remote_copy(
                src_ref=sl, dst_ref=sl,
                send_sem=fs_sems.at[f], recv_sem=fr_sems.at[f],
                device_id=nbr_y, device_id_type=pl.DeviceIdType.MESH,
            )

        def frecv(f):
            sl = out_ref.at[:, pl.ds(nbr_col0 + (16 - q0 + f) * TILE, TILE)]
            return pltpu.make_async_remote_copy(
                src_ref=sl, dst_ref=sl,
                send_sem=fs_sems.at[f], recv_sem=fr_sems.at[f],
                device_id=nbr_y, device_id_type=pl.DeviceIdType.MESH,
            )

        barrier_sem = pltpu.get_barrier_semaphore()
        for nb in (nbr_x, nbr_y):
            pl.semaphore_signal(barrier_sem, inc=1, device_id=nb,
                                device_id_type=pl.DeviceIdType.MESH)
        pl.semaphore_wait(barrier_sem, 2)

        w_load(0, 0).start()
        for j in range(n_loc):
            if j >= NSEND:
                f = j - NSEND
                xrecv(f).wait_recv()
                fwd(f).start()

            w_load(j, j % 2).wait()
            if j + 1 < n_loc:
                w_load(j + 1, (j + 1) % 2).start()
            logits = jnp.dot(x_ref[:, :], wtile[j % 2],
                             preferred_element_type=jnp.float32)
            mt = jnp.max(logits, axis=1, keepdims=True)
            e = jnp.exp(logits - mt)
            st = jnp.sum(e, axis=1, keepdims=True)
            mstat[:, j:j + 1] = mt
            sstat[:, j:j + 1] = st

            if j < NSEND:
                slot = j % NSLOT
                if j >= NSLOT:
                    xsend(j - NSLOT).wait_send()
                    cpo(j - NSLOT, send_ring, slot).wait()
                send_ring[slot] = e
                cpo(j, send_ring, slot).start()
                xsend(j).start()
            else:
                slot = j % 2
                if j >= NSEND + 2:
                    cpo(j - 2, fast_ring, slot).wait()
                fast_ring[slot] = e
                cpo(j, fast_ring, slot).start()

        for j in range(NSEND - NSLOT, NSEND):
            cpo(j, send_ring, j % NSLOT).wait()
        for j in range(n_loc - 2, n_loc):
            cpo(j, fast_ring, j % 2).wait()

        ms_send[:, 0:128] = mstat[:, :]
        ms_send[:, 128:256] = sstat[:, :]
        ms_rdma = pltpu.make_async_remote_copy(
            src_ref=ms_send, dst_ref=ms_recv,
            send_sem=ms_sems.at[0], recv_sem=ms_sems.at[1],
            device_id=nbr_x, device_id_type=pl.DeviceIdType.MESH)
        ms_rdma.start()
        ms_rdma.wait()

        m_mine_j = ms_send[:, 0:32]
        s_mine_j = ms_send[:, 128:160]
        m_nbr_j = ms_recv[:, 0:32]
        s_nbr_j = ms_recv[:, 128:160]
        m_all = jnp.maximum(jnp.max(m_mine_j, axis=1, keepdims=True),
                            jnp.max(m_nbr_j, axis=1, keepdims=True))
        s_all = (jnp.sum(jnp.exp(m_mine_j - m_all) * s_mine_j,
                         axis=1, keepdims=True)
                 + jnp.sum(jnp.exp(m_nbr_j - m_all) * s_nbr_j,
                           axis=1, keepdims=True))
        inv_s = 1.0 / s_all

        def by_tile(a):
            swapped = jnp.concatenate([a[:, 16:32], a[:, 0:16]], axis=1)
            return jnp.where(my_y == 0, a, swapped)

        scale_ref[:, 0:32] = jnp.exp(by_tile(m_mine_j) - m_all) * inv_s
        scale_ref[:, 32:64] = jnp.exp(by_tile(m_nbr_j) - m_all) * inv_s

        def p2_in(col_base, t, slot):
            return pltpu.make_async_copy(
                out_ref.at[:, pl.ds(col_base + t * TILE, TILE)],
                fast_ring.at[slot], lsem.at[slot])

        def p2_out(col_base, t, slot):
            return pltpu.make_async_copy(
                otile.at[slot],
                out_ref.at[:, pl.ds(col_base + t * TILE, TILE)],
                osem.at[slot])

        def pass2_range(k0, col_base):
            p2_in(col_base, 0, 0).start()
            p2_in(col_base, 1, 1).start()
            for t in range(n_loc):
                slot = t % 2
                p2_in(col_base, t, slot).wait()
                val = fast_ring[slot] * scale_ref[:, k0 + t:k0 + t + 1]
                if t >= 2:
                    p2_out(col_base, t - 2, slot).wait()
                otile[slot] = val
                p2_out(col_base, t, slot).start()
                if t + 2 < n_loc:
                    p2_in(col_base, t + 2, slot).start()
            p2_out(col_base, n_loc - 2, 0).wait()
            p2_out(col_base, n_loc - 1, 1).wait()

        pass2_range(0, col0)

        for j in range(NSEND - NSLOT, NSEND):
            xsend(j).wait_send()
        for f in range(NSEND):
            fwd(f).wait_send()
        for f in range(NSEND):
            frecv(f).wait_recv()

        pass2_range(32, nbr_col0)

    return pl.pallas_call(
        body,
        out_shape=jax.ShapeDtypeStruct((T, V), jnp.float32),
        in_specs=[
            pl.BlockSpec(memory_space=pltpu.VMEM),
            pl.BlockSpec(memory_space=pl.ANY),
        ],
        out_specs=pl.BlockSpec(memory_space=pl.ANY),
        scratch_shapes=[
            pltpu.VMEM((2, D, TILE), jnp.float32),
            pltpu.VMEM((NSLOT, T, TILE), jnp.float32),
            pltpu.VMEM((2, T, TILE), jnp.float32),
            pltpu.VMEM((2, T, TILE), jnp.float32),
            pltpu.VMEM((T, 128), jnp.float32),
            pltpu.VMEM((T, 128), jnp.float32),
            pltpu.VMEM((T, 256), jnp.float32),
            pltpu.VMEM((T, 256), jnp.float32),
            pltpu.VMEM((T, 128), jnp.float32),
            pltpu.SemaphoreType.DMA((2,)),
            pltpu.SemaphoreType.DMA((2,)),
            pltpu.SemaphoreType.DMA((n_loc,)),
            pltpu.SemaphoreType.DMA((NSEND,)),
            pltpu.SemaphoreType.DMA((NSEND,)),
            pltpu.SemaphoreType.DMA((NSEND,)),
            pltpu.SemaphoreType.DMA((NSEND,)),
            pltpu.SemaphoreType.DMA((2,)),
        ],
        compiler_params=pltpu.CompilerParams(
            collective_id=0, vmem_limit_bytes=96 * 1024 * 1024),
    )(x, W)


# device time: 377489 ns/iter; 3.3285x vs baseline; 1.5870x over previous
import jax
import jax.numpy as jnp
from jax import lax
from jax.experimental import pallas as pl
from jax.experimental.pallas import tpu as pltpu

TILE = 512
NSEND = 16
NSLOT = 12


def kernel(x, W):
    T, D = x.shape
    _, V_loc = W.shape
    V = 2 * V_loc
    n_loc = V_loc // TILE
    n_all = V // TILE

    def body(x_ref, w_ref, out_ref, e_hbm, xb_ref, wtile, send_ring,
             fast_ring, otile, mstat, sstat, ms_send, ms_recv, scale_ref,
             lsem, osem, cposem, xs_sems, xr_sems, fs_sems, fr_sems,
             ms_sems):
        my_x = lax.axis_index("x")
        my_y = lax.axis_index("y")
        nbr_x = (1 - my_x, my_y)
        nbr_y = (my_x, 1 - my_y)
        col0 = my_x * V_loc
        nbr_col0 = (1 - my_x) * V_loc
        q0 = 16 * my_y

        def tile(j):
            return q0 + j if j < NSEND else j - q0

        def w_load(j, slot):
            return pltpu.make_async_copy(
                w_ref.at[:, pl.ds(tile(j) * TILE, TILE)], wtile.at[slot],
                lsem.at[slot])

        def cpo(j, ring, slot):
            return pltpu.make_async_copy(
                ring.at[slot],
                e_hbm.at[:, pl.ds(col0 + tile(j) * TILE, TILE)],
                cposem.at[j])

        def xsend(j):
            return pltpu.make_async_remote_copy(
                src_ref=send_ring.at[j % NSLOT],
                dst_ref=e_hbm.at[:, pl.ds(col0 + tile(j) * TILE, TILE)],
                send_sem=xs_sems.at[j],
                recv_sem=xr_sems.at[j],
                device_id=nbr_x,
                device_id_type=pl.DeviceIdType.MESH,
            )

        def xrecv(f):
            sl = e_hbm.at[:, pl.ds(nbr_col0 + (q0 + f) * TILE, TILE)]
            return pltpu.make_async_remote_copy(
                src_ref=sl, dst_ref=sl,
                send_sem=xs_sems.at[f], recv_sem=xr_sems.at[f],
                device_id=nbr_x, device_id_type=pl.DeviceIdType.MESH,
            )

        def fwd(f):
            sl = e_hbm.at[:, pl.ds(nbr_col0 + (q0 + f) * TILE, TILE)]
            return pltpu.make_async_remote_copy(
                src_ref=sl, dst_ref=sl,
                send_sem=fs_sems.at[f], recv_sem=fr_sems.at[f],
                device_id=nbr_y, device_id_type=pl.DeviceIdType.MESH,
            )

        def frecv(f):
            sl = e_hbm.at[:, pl.ds(nbr_col0 + (16 - q0 + f) * TILE, TILE)]
            return pltpu.make_async_remote_copy(
                src_ref=sl, dst_ref=sl,
                send_sem=fs_sems.at[f], recv_sem=fr_sems.at[f],
                device_id=nbr_y, device_id_type=pl.DeviceIdType.MESH,
            )

        barrier_sem = pltpu.get_barrier_semaphore()
        for nb in (nbr_x, nbr_y):
            pl.semaphore_signal(barrier_sem, inc=1, device_id=nb,
                                device_id_type=pl.DeviceIdType.MESH)
        pl.semaphore_wait(barrier_sem, 2)

        xb_ref[:, :] = x_ref[:, :].astype(jnp.bfloat16)
        w_load(0, 0).start()
        for j in range(n_loc):
            if j >= NSEND:
                f = j - NSEND
                xrecv(f).wait_recv()
                fwd(f).start()

            w_load(j, j % 2).wait()
            if j + 1 < n_loc:
                w_load(j + 1, (j + 1) % 2).start()
            logits = jnp.dot(xb_ref[:, :],
                             wtile[j % 2].astype(jnp.bfloat16),
                             preferred_element_type=jnp.float32)
            mt = jnp.max(logits, axis=1, keepdims=True)
            e = jnp.exp(logits - mt)
            st = jnp.sum(e, axis=1, keepdims=True)
            mstat[:, j:j + 1] = mt
            sstat[:, j:j + 1] = st
            eb = e.astype(jnp.bfloat16)

            if j < NSEND:
                slot = j % NSLOT
                if j >= NSLOT:
                    xsend(j - NSLOT).wait_send()
                    cpo(j - NSLOT, send_ring, slot).wait()
                send_ring[slot] = eb
                cpo(j, send_ring, slot).start()
                xsend(j).start()
            else:
                slot = j % 2
                if j >= NSEND + 2:
                    cpo(j - 2, fast_ring, slot).wait()
                fast_ring[slot] = eb
                cpo(j, fast_ring, slot).start()

        for j in range(NSEND - NSLOT, NSEND):
            cpo(j, send_ring, j % NSLOT).wait()
        for j in range(n_loc - 2, n_loc):
            cpo(j, fast_ring, j % 2).wait()

        ms_send[:, 0:128] = mstat[:, :]
        ms_send[:, 128:256] = sstat[:, :]
        ms_rdma = pltpu.make_async_remote_copy(
            src_ref=ms_send, dst_ref=ms_recv,
            send_sem=ms_sems.at[0], recv_sem=ms_sems.at[1],
            device_id=nbr_x, device_id_type=pl.DeviceIdType.MESH)
        ms_rdma.start()
        ms_rdma.wait()

        m_mine_j = ms_send[:, 0:32]
        s_mine_j = ms_send[:, 128:160]
        m_nbr_j = ms_recv[:, 0:32]
        s_nbr_j = ms_recv[:, 128:160]
        m_all = jnp.maximum(jnp.max(m_mine_j, axis=1, keepdims=True),
                            jnp.max(m_nbr_j, axis=1, keepdims=True))
        s_all = (jnp.sum(jnp.exp(m_mine_j - m_all) * s_mine_j,
                         axis=1, keepdims=True)
                 + jnp.sum(jnp.exp(m_nbr_j - m_all) * s_nbr_j,
                           axis=1, keepdims=True))
        inv_s = 1.0 / s_all

        def by_tile(a):
            swapped = jnp.concatenate([a[:, 16:32], a[:, 0:16]], axis=1)
            return jnp.where(my_y == 0, a, swapped)

        scale_ref[:, 0:32] = jnp.exp(by_tile(m_mine_j) - m_all) * inv_s
        scale_ref[:, 32:64] = jnp.exp(by_tile(m_nbr_j) - m_all) * inv_s

        def p2_in(col_base, t, slot):
            return pltpu.make_async_copy(
                e_hbm.at[:, pl.ds(col_base + t * TILE, TILE)],
                fast_ring.at[slot], lsem.at[slot])

        def p2_out(col_base, t, slot):
            return pltpu.make_async_copy(
                otile.at[slot],
                out_ref.at[:, pl.ds(col_base + t * TILE, TILE)],
                osem.at[slot])

        def pass2_range(k0, col_base):
            p2_in(col_base, 0, 0).start()
            p2_in(col_base, 1, 1).start()
            for t in range(n_loc):
                slot = t % 2
                p2_in(col_base, t, slot).wait()
                val = (fast_ring[slot].astype(jnp.float32)
                       * scale_ref[:, k0 + t:k0 + t + 1])
                if t >= 2:
                    p2_out(col_base, t - 2, slot).wait()
                otile[slot] = val
                p2_out(col_base, t, slot).start()
                if t + 2 < n_loc:
                    p2_in(col_base, t + 2, slot).start()
            p2_out(col_base, n_loc - 2, 0).wait()
            p2_out(col_base, n_loc - 1, 1).wait()

        pass2_range(0, col0)

        for j in range(NSEND - NSLOT, NSEND):
            xsend(j).wait_send()
        for f in range(NSEND):
            fwd(f).wait_send()
        for f in range(NSEND):
            frecv(f).wait_recv()

        pass2_range(32, nbr_col0)

    out, _ = pl.pallas_call(
        body,
        out_shape=[
            jax.ShapeDtypeStruct((T, V), jnp.float32),
            jax.ShapeDtypeStruct((T, V), jnp.bfloat16),
        ],
        in_specs=[
            pl.BlockSpec(memory_space=pltpu.VMEM),
            pl.BlockSpec(memory_space=pl.ANY),
        ],
        out_specs=[
            pl.BlockSpec(memory_space=pl.ANY),
            pl.BlockSpec(memory_space=pl.ANY),
        ],
        scratch_shapes=[
            pltpu.VMEM((T, D), jnp.bfloat16),
            pltpu.VMEM((2, D, TILE), jnp.float32),
            pltpu.VMEM((NSLOT, T, TILE), jnp.bfloat16),
            pltpu.VMEM((2, T, TILE), jnp.bfloat16),
            pltpu.VMEM((2, T, TILE), jnp.float32),
            pltpu.VMEM((T, 128), jnp.float32),
            pltpu.VMEM((T, 128), jnp.float32),
            pltpu.VMEM((T, 256), jnp.float32),
            pltpu.VMEM((T, 256), jnp.float32),
            pltpu.VMEM((T, 128), jnp.float32),
            pltpu.SemaphoreType.DMA((2,)),
            pltpu.SemaphoreType.DMA((2,)),
            pltpu.SemaphoreType.DMA((n_loc,)),
            pltpu.SemaphoreType.DMA((NSEND,)),
            pltpu.SemaphoreType.DMA((NSEND,)),
            pltpu.SemaphoreType.DMA((NSEND,)),
            pltpu.SemaphoreType.DMA((NSEND,)),
            pltpu.SemaphoreType.DMA((2,)),
        ],
        compiler_params=pltpu.CompilerParams(
            collective_id=0, vmem_limit_bytes=96 * 1024 * 1024),
    )(x, W)
    return out


# device time: 365725 ns/iter; 3.4355x vs baseline; 1.0322x over previous
import jax
import jax.numpy as jnp
from jax import lax
from jax.experimental import pallas as pl
from jax.experimental.pallas import tpu as pltpu

TILE = 512
NSEND = 16
NSLOT = 12


def kernel(x, W):
    T, D = x.shape
    _, V_loc = W.shape
    V = 2 * V_loc
    n_loc = V_loc // TILE
    n_all = V // TILE

    def body(xb_ref, w_ref, out_ref, e_hbm, wtile, send_ring,
             fast_ring, p2ring, otile, mstat, sstat, ms_send, ms_recv,
             scale_ref,
             lsem, osem, cposem, xs_sems, xr_sems, fs_sems, fr_sems,
             ms_sems):
        my_x = lax.axis_index("x")
        my_y = lax.axis_index("y")
        nbr_x = (1 - my_x, my_y)
        nbr_y = (my_x, 1 - my_y)
        col0 = my_x * V_loc
        nbr_col0 = (1 - my_x) * V_loc
        q0 = 16 * my_y

        def tile(j):
            return q0 + j if j < NSEND else j - q0

        def w_load(j, slot):
            return pltpu.make_async_copy(
                w_ref.at[:, pl.ds(tile(j) * TILE, TILE)], wtile.at[slot],
                lsem.at[slot])

        def cpo(j, ring, slot):
            return pltpu.make_async_copy(
                ring.at[slot],
                e_hbm.at[:, pl.ds(col0 + tile(j) * TILE, TILE)],
                cposem.at[j])

        def xsend(j):
            return pltpu.make_async_remote_copy(
                src_ref=send_ring.at[j % NSLOT],
                dst_ref=e_hbm.at[:, pl.ds(col0 + tile(j) * TILE, TILE)],
                send_sem=xs_sems.at[j],
                recv_sem=xr_sems.at[j],
                device_id=nbr_x,
                device_id_type=pl.DeviceIdType.MESH,
            )

        def xrecv(f):
            sl = e_hbm.at[:, pl.ds(nbr_col0 + (q0 + f) * TILE, TILE)]
            return pltpu.make_async_remote_copy(
                src_ref=sl, dst_ref=sl,
                send_sem=xs_sems.at[f], recv_sem=xr_sems.at[f],
                device_id=nbr_x, device_id_type=pl.DeviceIdType.MESH,
            )

        def fwd(f):
            sl = e_hbm.at[:, pl.ds(nbr_col0 + (q0 + f) * TILE, TILE)]
            return pltpu.make_async_remote_copy(
                src_ref=sl, dst_ref=sl,
                send_sem=fs_sems.at[f], recv_sem=fr_sems.at[f],
                device_id=nbr_y, device_id_type=pl.DeviceIdType.MESH,
            )

        def frecv(f):
            sl = e_hbm.at[:, pl.ds(nbr_col0 + (16 - q0 + f) * TILE, TILE)]
            return pltpu.make_async_remote_copy(
                src_ref=sl, dst_ref=sl,
                send_sem=fs_sems.at[f], recv_sem=fr_sems.at[f],
                device_id=nbr_y, device_id_type=pl.DeviceIdType.MESH,
            )

        barrier_sem = pltpu.get_barrier_semaphore()
        for nb in (nbr_x, nbr_y):
            pl.semaphore_signal(barrier_sem, inc=1, device_id=nb,
                                device_id_type=pl.DeviceIdType.MESH)
        pl.semaphore_wait(barrier_sem, 2)

        w_load(0, 0).start()
        for j in range(n_loc):
            if j >= NSEND:
                f = j - NSEND
                xrecv(f).wait_recv()
                fwd(f).start()

            w_load(j, j % 2).wait()
            if j + 1 < n_loc:
                w_load(j + 1, (j + 1) % 2).start()
            logits = jnp.dot(xb_ref[:, :],
                             wtile[j % 2].astype(jnp.bfloat16),
                             preferred_element_type=jnp.float32)
            mt = jnp.max(logits, axis=1, keepdims=True)
            e = jnp.exp(logits - mt)
            st = jnp.sum(e, axis=1, keepdims=True)
            mstat[:, j:j + 1] = mt
            sstat[:, j:j + 1] = st
            eb = e.astype(jnp.bfloat16)

            if j < NSEND:
                slot = j % NSLOT
                if j >= NSLOT:
                    xsend(j - NSLOT).wait_send()
                    cpo(j - NSLOT, send_ring, slot).wait()
                send_ring[slot] = eb
                cpo(j, send_ring, slot).start()
                xsend(j).start()
            else:
                slot = j % 2
                if j >= NSEND + 2:
                    cpo(j - 2, fast_ring, slot).wait()
                fast_ring[slot] = eb
                cpo(j, fast_ring, slot).start()

        for j in range(NSEND - NSLOT, NSEND):
            cpo(j, send_ring, j % NSLOT).wait()
        for j in range(n_loc - 2, n_loc):
            cpo(j, fast_ring, j % 2).wait()

        ms_send[:, 0:128] = mstat[:, :]
        ms_send[:, 128:256] = sstat[:, :]
        ms_rdma = pltpu.make_async_remote_copy(
            src_ref=ms_send, dst_ref=ms_recv,
            send_sem=ms_sems.at[0], recv_sem=ms_sems.at[1],
            device_id=nbr_x, device_id_type=pl.DeviceIdType.MESH)
        ms_rdma.start()
        ms_rdma.wait()

        m_mine_j = ms_send[:, 0:32]
        s_mine_j = ms_send[:, 128:160]
        m_nbr_j = ms_recv[:, 0:32]
        s_nbr_j = ms_recv[:, 128:160]
        m_all = jnp.maximum(jnp.max(m_mine_j, axis=1, keepdims=True),
                            jnp.max(m_nbr_j, axis=1, keepdims=True))
        s_all = (jnp.sum(jnp.exp(m_mine_j - m_all) * s_mine_j,
                         axis=1, keepdims=True)
                 + jnp.sum(jnp.exp(m_nbr_j - m_all) * s_nbr_j,
                           axis=1, keepdims=True))
        inv_s = 1.0 / s_all

        def by_tile(a):
            swapped = jnp.concatenate([a[:, 16:32], a[:, 0:16]], axis=1)
            return jnp.where(my_y == 0, a, swapped)

        scale_ref[:, 0:32] = jnp.exp(by_tile(m_mine_j) - m_all) * inv_s
        scale_ref[:, 32:64] = jnp.exp(by_tile(m_nbr_j) - m_all) * inv_s

        TILE2 = 4 * TILE
        n2 = V_loc // TILE2

        def p2_in(col_base, t, slot):
            return pltpu.make_async_copy(
                e_hbm.at[:, pl.ds(col_base + t * TILE2, TILE2)],
                p2ring.at[slot], lsem.at[slot])

        def p2_out(col_base, t, slot):
            return pltpu.make_async_copy(
                otile.at[slot],
                out_ref.at[:, pl.ds(col_base + t * TILE2, TILE2)],
                osem.at[slot])

        def pass2_range(k0, col_base):
            p2_in(col_base, 0, 0).start()
            p2_in(col_base, 1, 1).start()
            for t in range(n2):
                slot = t % 2
                p2_in(col_base, t, slot).wait()
                if t >= 2:
                    p2_out(col_base, t - 2, slot).wait()
                for c in range(4):
                    k = k0 + 4 * t + c
                    otile[slot, :, c * TILE:(c + 1) * TILE] = (
                        p2ring[slot][:, c * TILE:(c + 1) * TILE]
                        .astype(jnp.float32)
                        * scale_ref[:, k:k + 1])
                p2_out(col_base, t, slot).start()
                if t + 2 < n2:
                    p2_in(col_base, t + 2, slot).start()
            p2_out(col_base, n2 - 2, 0).wait()
            p2_out(col_base, n2 - 1, 1).wait()

        pass2_range(0, col0)

        for j in range(NSEND - NSLOT, NSEND):
            xsend(j).wait_send()
        for f in range(NSEND):
            fwd(f).wait_send()
        for f in range(NSEND):
            frecv(f).wait_recv()

        pass2_range(32, nbr_col0)

    out, _ = pl.pallas_call(
        body,
        out_shape=[
            jax.ShapeDtypeStruct((T, V), jnp.float32),
            jax.ShapeDtypeStruct((T, V), jnp.bfloat16),
        ],
        in_specs=[
            pl.BlockSpec(memory_space=pltpu.VMEM),
            pl.BlockSpec(memory_space=pl.ANY),
        ],
        out_specs=[
            pl.BlockSpec(memory_space=pl.ANY),
            pl.BlockSpec(memory_space=pl.ANY),
        ],
        scratch_shapes=[
            pltpu.VMEM((2, D, TILE), jnp.float32),
            pltpu.VMEM((NSLOT, T, TILE), jnp.bfloat16),
            pltpu.VMEM((2, T, TILE), jnp.bfloat16),
            pltpu.VMEM((2, T, 4 * TILE), jnp.bfloat16),
            pltpu.VMEM((2, T, 4 * TILE), jnp.float32),
            pltpu.VMEM((T, 128), jnp.float32),
            pltpu.VMEM((T, 128), jnp.float32),
            pltpu.VMEM((T, 256), jnp.float32),
            pltpu.VMEM((T, 256), jnp.float32),
            pltpu.VMEM((T, 128), jnp.float32),
            pltpu.SemaphoreType.DMA((2,)),
            pltpu.SemaphoreType.DMA((2,)),
            pltpu.SemaphoreType.DMA((n_loc,)),
            pltpu.SemaphoreType.DMA((NSEND,)),
            pltpu.SemaphoreType.DMA((NSEND,)),
            pltpu.SemaphoreType.DMA((NSEND,)),
            pltpu.SemaphoreType.DMA((NSEND,)),
            pltpu.SemaphoreType.DMA((2,)),
        ],
        compiler_params=pltpu.CompilerParams(
            collective_id=0, vmem_limit_bytes=96 * 1024 * 1024),
    )(x.astype(jnp.bfloat16), W)
    return out


# device time: 365517 ns/iter; 3.4375x vs baseline; 1.0006x over previous
import jax
import jax.numpy as jnp
from jax import lax
from jax.experimental import pallas as pl
from jax.experimental.pallas import tpu as pltpu

TILE = 512
NSEND = 16
NSLOT = 12


def kernel(x, W):
    T, D = x.shape
    _, V_loc = W.shape
    V = 2 * V_loc
    n_loc = V_loc // TILE
    n_all = V // TILE

    def body(xb_ref, w_ref, out_ref, e_hbm, wtile, send_ring,
             fast_ring, p2ring, otile, mstat, sstat, ms_send, ms_recv,
             scale_ref,
             lsem, osem, cposem, xs_sems, xr_sems, fs_sems, fr_sems,
             ms_sems):
        my_x = lax.axis_index("x")
        my_y = lax.axis_index("y")
        nbr_x = (1 - my_x, my_y)
        nbr_y = (my_x, 1 - my_y)
        col0 = my_x * V_loc
        nbr_col0 = (1 - my_x) * V_loc
        my_base = my_x * n_loc
        nbr_base = (1 - my_x) * n_loc
        q0 = 16 * my_y

        def tile(j):
            return q0 + j if j < NSEND else j - q0

        def w_load(j, slot):
            return pltpu.make_async_copy(
                w_ref.at[:, pl.ds(tile(j) * TILE, TILE)], wtile.at[slot],
                lsem.at[slot])

        def cpo(j, ring, slot):
            return pltpu.make_async_copy(
                ring.at[slot], e_hbm.at[my_base + tile(j)], cposem.at[j])

        def xsend(j):
            return pltpu.make_async_remote_copy(
                src_ref=send_ring.at[j % NSLOT],
                dst_ref=e_hbm.at[my_base + tile(j)],
                send_sem=xs_sems.at[j],
                recv_sem=xr_sems.at[j],
                device_id=nbr_x,
                device_id_type=pl.DeviceIdType.MESH,
            )

        def xrecv(f):
            sl = e_hbm.at[nbr_base + q0 + f]
            return pltpu.make_async_remote_copy(
                src_ref=sl, dst_ref=sl,
                send_sem=xs_sems.at[f], recv_sem=xr_sems.at[f],
                device_id=nbr_x, device_id_type=pl.DeviceIdType.MESH,
            )

        def fwd(f):
            sl = e_hbm.at[nbr_base + q0 + f]
            return pltpu.make_async_remote_copy(
                src_ref=sl, dst_ref=sl,
                send_sem=fs_sems.at[f], recv_sem=fr_sems.at[f],
                device_id=nbr_y, device_id_type=pl.DeviceIdType.MESH,
            )

        def frecv(f):
            sl = e_hbm.at[nbr_base + 16 - q0 + f]
            return pltpu.make_async_remote_copy(
                src_ref=sl, dst_ref=sl,
                send_sem=fs_sems.at[f], recv_sem=fr_sems.at[f],
                device_id=nbr_y, device_id_type=pl.DeviceIdType.MESH,
            )

        barrier_sem = pltpu.get_barrier_semaphore()
        for nb in (nbr_x, nbr_y):
            pl.semaphore_signal(barrier_sem, inc=1, device_id=nb,
                                device_id_type=pl.DeviceIdType.MESH)
        pl.semaphore_wait(barrier_sem, 2)

        w_load(0, 0).start()
        for j in range(n_loc):
            if j >= NSEND:
                f = j - NSEND
                xrecv(f).wait_recv()
                fwd(f).start()

            w_load(j, j % 2).wait()
            if j + 1 < n_loc:
                w_load(j + 1, (j + 1) % 2).start()
            logits = jnp.dot(xb_ref[:, :],
                             wtile[j % 2].astype(jnp.bfloat16),
                             preferred_element_type=jnp.float32)
            mt = jnp.max(logits, axis=1, keepdims=True)
            e = jnp.exp(logits - mt)
            st = jnp.sum(e, axis=1, keepdims=True)
            mstat[:, j:j + 1] = mt
            sstat[:, j:j + 1] = st
            eb = e.astype(jnp.bfloat16)

            if j < NSEND:
                slot = j % NSLOT
                if j >= NSLOT:
                    xsend(j - NSLOT).wait_send()
                    cpo(j - NSLOT, send_ring, slot).wait()
                send_ring[slot] = eb
                cpo(j, send_ring, slot).start()
                xsend(j).start()
            else:
                slot = j % 2
                if j >= NSEND + 2:
                    cpo(j - 2, fast_ring, slot).wait()
                fast_ring[slot] = eb
                cpo(j, fast_ring, slot).start()

        for j in range(NSEND - NSLOT, NSEND):
            cpo(j, send_ring, j % NSLOT).wait()
        for j in range(n_loc - 2, n_loc):
            cpo(j, fast_ring, j % 2).wait()

        ms_send[:, 0:128] = mstat[:, :]
        ms_send[:, 128:256] = sstat[:, :]
        ms_rdma = pltpu.make_async_remote_copy(
            src_ref=ms_send, dst_ref=ms_recv,
            send_sem=ms_sems.at[0], recv_sem=ms_sems.at[1],
            device_id=nbr_x, device_id_type=pl.DeviceIdType.MESH)
        ms_rdma.start()
        ms_rdma.wait()

        m_mine_j = ms_send[:, 0:32]
        s_mine_j = ms_send[:, 128:160]
        m_nbr_j = ms_recv[:, 0:32]
        s_nbr_j = ms_recv[:, 128:160]
        m_all = jnp.maximum(jnp.max(m_mine_j, axis=1, keepdims=True),
                            jnp.max(m_nbr_j, axis=1, keepdims=True))
        s_all = (jnp.sum(jnp.exp(m_mine_j - m_all) * s_mine_j,
                         axis=1, keepdims=True)
                 + jnp.sum(jnp.exp(m_nbr_j - m_all) * s_nbr_j,
                           axis=1, keepdims=True))
        inv_s = 1.0 / s_all

        def by_tile(a):
            swapped = jnp.concatenate([a[:, 16:32], a[:, 0:16]], axis=1)
            return jnp.where(my_y == 0, a, swapped)

        scale_ref[:, 0:32] = jnp.exp(by_tile(m_mine_j) - m_all) * inv_s
        scale_ref[:, 32:64] = jnp.exp(by_tile(m_nbr_j) - m_all) * inv_s

        TILE2 = 4 * TILE
        n2 = V_loc // TILE2

        def p2_in(tile_base, t, slot):
            return pltpu.make_async_copy(
                e_hbm.at[pl.ds(tile_base + 4 * t, 4)],
                p2ring.at[slot], lsem.at[slot])

        def p2_out(col_base, t, slot):
            return pltpu.make_async_copy(
                otile.at[slot],
                out_ref.at[:, pl.ds(col_base + t * TILE2, TILE2)],
                osem.at[slot])

        def pass2_range(k0, tile_base, col_base):
            p2_in(tile_base, 0, 0).start()
            p2_in(tile_base, 1, 1).start()
            for t in range(n2):
                slot = t % 2
                p2_in(tile_base, t, slot).wait()
                if t >= 2:
                    p2_out(col_base, t - 2, slot).wait()
                for c in range(4):
                    k = k0 + 4 * t + c
                    otile[slot, :, c * TILE:(c + 1) * TILE] = (
                        p2ring[slot, c].astype(jnp.float32)
                        * scale_ref[:, k:k + 1])
                p2_out(col_base, t, slot).start()
                if t + 2 < n2:
                    p2_in(tile_base, t + 2, slot).start()
            p2_out(col_base, n2 - 2, 0).wait()
            p2_out(col_base, n2 - 1, 1).wait()

        pass2_range(0, my_base, col0)

        for j in range(NSEND - NSLOT, NSEND):
            xsend(j).wait_send()
        for f in range(NSEND):
            fwd(f).wait_send()
        for f in range(NSEND):
            frecv(f).wait_recv()

        pass2_range(32, nbr_base, nbr_col0)

    out, _ = pl.pallas_call(
        body,
        out_shape=[
            jax.ShapeDtypeStruct((T, V), jnp.float32),
            jax.ShapeDtypeStruct((n_all, T, TILE), jnp.bfloat16),
        ],
        in_specs=[
            pl.BlockSpec(memory_space=pltpu.VMEM),
            pl.BlockSpec(memory_space=pl.ANY),
        ],
        out_specs=[
            pl.BlockSpec(memory_space=pl.ANY),
            pl.BlockSpec(memory_space=pl.ANY),
        ],
        scratch_shapes=[
            pltpu.VMEM((2, D, TILE), jnp.float32),
            pltpu.VMEM((NSLOT, T, TILE), jnp.bfloat16),
            pltpu.VMEM((2, T, TILE), jnp.bfloat16),
            pltpu.VMEM((2, 4, T, TILE), jnp.bfloat16),
            pltpu.VMEM((2, T, 4 * TILE), jnp.float32),
            pltpu.VMEM((T, 128), jnp.float32),
            pltpu.VMEM((T, 128), jnp.float32),
            pltpu.VMEM((T, 256), jnp.float32),
            pltpu.VMEM((T, 256), jnp.float32),
            pltpu.VMEM((T, 128), jnp.float32),
            pltpu.SemaphoreType.DMA((2,)),
            pltpu.SemaphoreType.DMA((2,)),
            pltpu.SemaphoreType.DMA((n_loc,)),
            pltpu.SemaphoreType.DMA((NSEND,)),
            pltpu.SemaphoreType.DMA((NSEND,)),
            pltpu.SemaphoreType.DMA((NSEND,)),
            pltpu.SemaphoreType.DMA((NSEND,)),
            pltpu.SemaphoreType.DMA((2,)),
        ],
        compiler_params=pltpu.CompilerParams(
            collective_id=0, vmem_limit_bytes=96 * 1024 * 1024),
    )(x.astype(jnp.bfloat16), W)
    return out
